# baseline (speedup 1.0000x reference)
import functools

import jax
import jax.numpy as jnp
from jax import lax
from jax.experimental import pallas as pl
from jax.experimental.pallas import tpu as pltpu
from jax.experimental.pallas import tpu_sc as plsc

_NC = 2
_NS = 16
_NW = _NC * _NS
_LANES = 16


def kernel(x, token_table, pos_table):
    B, L = x.shape
    V, E = token_table.shape
    assert B % _NW == 0 and L % 2 == 0 and E % _LANES == 0
    H = 2
    LH = L // H
    n = B // _NW

    x32 = x.astype(jnp.int32).reshape(_NW, n, H, LH)
    token_table = jax.lax.optimization_barrier(token_table.T).T

    mesh = plsc.VectorSubcoreMesh(core_axis_name="c", subcore_axis_name="s")

    @functools.partial(
        pl.kernel,
        out_type=jax.ShapeDtypeStruct((B, L, E), jnp.float32),
        mesh=mesh,
        compiler_params=pltpu.CompilerParams(use_tc_tiling_on_sc=False),
        scratch_types=[
            pltpu.VMEM((n, H, LH), jnp.int32),
            pltpu.VMEM((L, E), jnp.float32),
            pltpu.VMEM((L, E), jnp.float32),
            pltpu.VMEM((L, E), jnp.float32),
            pltpu.SemaphoreType.DMA,
            pltpu.SemaphoreType.DMA,
            pltpu.SemaphoreType.DMA,
            pltpu.SemaphoreType.DMA,
        ],
    )
    def emb_kernel(x_hbm, tok_hbm, pos_hbm, out_hbm,
                   idx_v, buf0, buf1, pos_v, g0, g1, o0, o1):
        wid = lax.axis_index("s") * _NC + lax.axis_index("c")
        b0 = wid * n
        bufs, gsems, osems = (buf0, buf1), (g0, g1), (o0, o1)

        pltpu.sync_copy(x_hbm.at[wid], idx_v)
        pltpu.sync_copy(pos_hbm, pos_v)

        def start_gather(i):
            bb = bufs[i % 2]
            c0 = pltpu.async_copy(tok_hbm.at[idx_v.at[i, 0]],
                                  bb.at[pl.ds(0, LH)], gsems[i % 2])
            c1 = pltpu.async_copy(tok_hbm.at[idx_v.at[i, 1]],
                                  bb.at[pl.ds(LH, LH)], gsems[i % 2])
            return c0, c1

        pend_g = {0: start_gather(0)}
        pend_o = {}
        for i in range(n):
            bsel = i % 2
            if i + 1 < n:
                if i - 1 in pend_o:
                    pend_o.pop(i - 1).wait()
                pend_g[i + 1] = start_gather(i + 1)
            c0, c1 = pend_g.pop(i)
            c0.wait()
            c1.wait()

            def add_row(r, c, _b=bufs[bsel]):
                for k in range(E // _LANES):
                    sl = pl.ds(k * _LANES, _LANES)
                    _b[r, sl] = _b[r, sl] + pos_v[r, sl]
                return c

            lax.fori_loop(0, L, add_row, 0, unroll=2)
            pend_o[i] = pltpu.async_copy(bufs[bsel], out_hbm.at[b0 + i],
                                         osems[bsel])
        for i in sorted(pend_o):
            pend_o.pop(i).wait()

    return emb_kernel(x32, token_table, pos_table)


# speedup vs baseline: 1.0297x; 1.0297x over previous
import functools

import jax
import jax.numpy as jnp
from jax import lax
from jax.experimental import pallas as pl
from jax.experimental.pallas import tpu as pltpu
from jax.experimental.pallas import tpu_sc as plsc

_NC = 2
_NS = 16
_NW = _NC * _NS
_LANES = 16


def kernel(x, token_table, pos_table):
    B, L = x.shape
    V, E = token_table.shape
    assert B % _NW == 0 and L % 2 == 0 and E % _LANES == 0
    H = 2
    LH = L // H
    n = B // _NW

    x32 = x.astype(jnp.int32).reshape(_NW, n, H, LH)

    mesh = plsc.VectorSubcoreMesh(core_axis_name="c", subcore_axis_name="s")

    @functools.partial(
        pl.kernel,
        out_type=jax.ShapeDtypeStruct((B, L, E), jnp.float32),
        mesh=mesh,
        compiler_params=pltpu.CompilerParams(use_tc_tiling_on_sc=False),
        scratch_types=[
            pltpu.VMEM((n, H, LH), jnp.int32),
            pltpu.VMEM((3, L, E), jnp.float32),
            pltpu.VMEM((L, E), jnp.float32),
            pltpu.SemaphoreType.DMA,
            pltpu.SemaphoreType.DMA,
            pltpu.SemaphoreType.DMA,
            pltpu.SemaphoreType.DMA,
            pltpu.SemaphoreType.DMA,
            pltpu.SemaphoreType.DMA,
        ],
    )
    def emb_kernel(x_hbm, tok_hbm, pos_hbm, out_hbm,
                   idx_v, bufs, pos_v, g0, g1, g2, o0, o1, o2):
        wid = lax.axis_index("s") * _NC + lax.axis_index("c")
        b0 = wid * n
        gsems, osems = (g0, g1, g2), (o0, o1, o2)
        D = 3

        pltpu.sync_copy(x_hbm.at[wid], idx_v)
        pltpu.sync_copy(pos_hbm, pos_v)

        def start_gather(i):
            bb = bufs.at[i % D]
            c0 = pltpu.async_copy(tok_hbm.at[idx_v.at[i, 0]],
                                  bb.at[pl.ds(0, LH)], gsems[i % D])
            c1 = pltpu.async_copy(tok_hbm.at[idx_v.at[i, 1]],
                                  bb.at[pl.ds(LH, LH)], gsems[i % D])
            return c0, c1

        pend_g = {0: start_gather(0), 1: start_gather(1)}
        pend_o = {}
        for i in range(n):
            bsel = i % D
            if i + 2 < n:
                if i - 1 in pend_o:
                    pend_o.pop(i - 1).wait()
                pend_g[i + 2] = start_gather(i + 2)
            c0, c1 = pend_g.pop(i)
            c0.wait()
            c1.wait()

            def add_row(r, c, _i=bsel):
                for k in range(E // _LANES):
                    sl = pl.ds(k * _LANES, _LANES)
                    bufs[_i, r, sl] = bufs[_i, r, sl] + pos_v[r, sl]
                return c

            lax.fori_loop(0, L, add_row, 0, unroll=4)
            pend_o[i] = pltpu.async_copy(bufs.at[bsel], out_hbm.at[b0 + i],
                                         osems[bsel])
        for i in sorted(pend_o):
            pend_o.pop(i).wait()

    return emb_kernel(x32, token_table, pos_table)


# speedup vs baseline: 1.0491x; 1.0188x over previous
import functools

import jax
import jax.numpy as jnp
from jax import lax
from jax.experimental import pallas as pl
from jax.experimental.pallas import tpu as pltpu
from jax.experimental.pallas import tpu_sc as plsc

_NC = 2
_NS = 16
_NW = _NC * _NS
_LANES = 16


def kernel(x, token_table, pos_table):
    B, L = x.shape
    V, E = token_table.shape
    assert B % _NW == 0 and L % 2 == 0 and E % _LANES == 0
    H = 2
    LH = L // H
    n = B // _NW

    x32 = x.astype(jnp.int32).reshape(_NW, n, H, LH)

    mesh = plsc.VectorSubcoreMesh(core_axis_name="c", subcore_axis_name="s")

    @functools.partial(
        pl.kernel,
        out_type=jax.ShapeDtypeStruct((B, L, E), jnp.float32),
        mesh=mesh,
        compiler_params=pltpu.CompilerParams(use_tc_tiling_on_sc=False),
        scratch_types=[
            pltpu.VMEM((n, H, LH), jnp.int32),
            pltpu.VMEM((4, L, E), jnp.float32),
            pltpu.VMEM((L, E), jnp.float32),
            pltpu.SemaphoreType.DMA,
            pltpu.SemaphoreType.DMA,
            pltpu.SemaphoreType.DMA,
            pltpu.SemaphoreType.DMA,
            pltpu.SemaphoreType.DMA,
            pltpu.SemaphoreType.DMA,
            pltpu.SemaphoreType.DMA,
            pltpu.SemaphoreType.DMA,
        ],
    )
    def emb_kernel(x_hbm, tok_hbm, pos_hbm, out_hbm,
                   idx_v, bufs, pos_v, g0, g1, g2, g3, o0, o1, o2, o3):
        wid = lax.axis_index("s") * _NC + lax.axis_index("c")
        b0 = wid * n
        gsems, osems = (g0, g1, g2, g3), (o0, o1, o2, o3)
        D = 4

        pltpu.sync_copy(x_hbm.at[wid], idx_v)
        pltpu.sync_copy(pos_hbm, pos_v)

        def start_gather(i):
            bb = bufs.at[i % D]
            c0 = pltpu.async_copy(tok_hbm.at[idx_v.at[i, 0]],
                                  bb.at[pl.ds(0, LH)], gsems[i % D])
            c1 = pltpu.async_copy(tok_hbm.at[idx_v.at[i, 1]],
                                  bb.at[pl.ds(LH, LH)], gsems[i % D])
            return c0, c1

        pend_g = {0: start_gather(0), 1: start_gather(1), 2: start_gather(2)}
        pend_o = {}
        for i in range(n):
            bsel = i % D
            if i + 3 < n:
                if i - 1 in pend_o:
                    pend_o.pop(i - 1).wait()
                pend_g[i + 3] = start_gather(i + 3)
            c0, c1 = pend_g.pop(i)
            c0.wait()
            c1.wait()

            def add_row(r, c, _i=bsel):
                for k in range(E // _LANES):
                    sl = pl.ds(k * _LANES, _LANES)
                    bufs[_i, r, sl] = bufs[_i, r, sl] + pos_v[r, sl]
                return c

            lax.fori_loop(0, L, add_row, 0, unroll=4)
            pend_o[i] = pltpu.async_copy(bufs.at[bsel], out_hbm.at[b0 + i],
                                         osems[bsel])
        for i in sorted(pend_o):
            pend_o.pop(i).wait()

    return emb_kernel(x32, token_table, pos_table)
